# baseline (speedup 1.0000x reference)
import functools

import jax
import jax.numpy as jnp
from jax import lax
from jax.experimental import pallas as pl
from jax.experimental.pallas import tpu as pltpu
from jax.experimental.pallas import tpu_sc as plsc

VOCAB = 100000
EMBED = 64
BATCH = 4096
CTX = 20

_NC = 2
_NS = 16
_NW = _NC * _NS

_ROWS_PER_W = BATCH // _NW
_CHUNK_ROWS = 4
_CHUNK_IDX = _CHUNK_ROWS * CTX
_NCHUNKS = _ROWS_PER_W // _CHUNK_ROWS
_LANES = 16
_GROUPS = EMBED // _LANES


def _pool_body(idx_hbm, table_hbm, out_hbm, idx_v, rows_v, out_v, sem):
    wid = lax.axis_index("s") * _NC + lax.axis_index("c")
    idx_base = wid * (_ROWS_PER_W * CTX)
    row_base = wid * _ROWS_PER_W

    def chunk(c, carry):
        off = pl.multiple_of(idx_base + c * _CHUNK_IDX, 8)
        pltpu.sync_copy(idx_hbm.at[pl.ds(off, _CHUNK_IDX)], idx_v)
        pltpu.async_copy(table_hbm.at[idx_v], rows_v, sem).wait()
        for r in range(_CHUNK_ROWS):
            for g in range(_GROUPS):
                acc = rows_v[r * CTX, pl.ds(g * _LANES, _LANES)]
                for j in range(1, CTX):
                    acc = acc + rows_v[r * CTX + j, pl.ds(g * _LANES, _LANES)]
                out_v[r, pl.ds(g * _LANES, _LANES)] = acc * (1.0 / CTX)
        orow = pl.multiple_of(row_base + c * _CHUNK_ROWS, 4)
        pltpu.sync_copy(out_v, out_hbm.at[pl.ds(orow, _CHUNK_ROWS)])
        return carry

    lax.fori_loop(0, _NCHUNKS, chunk, 0)


_pool = functools.partial(
    pl.kernel,
    out_type=jax.ShapeDtypeStruct((BATCH, EMBED), jnp.float32),
    mesh=plsc.VectorSubcoreMesh(core_axis_name="c", subcore_axis_name="s"),
    scratch_types=[
        pltpu.VMEM((_CHUNK_IDX,), jnp.int32),
        pltpu.VMEM((_CHUNK_IDX, EMBED), jnp.float32),
        pltpu.VMEM((_CHUNK_ROWS, EMBED), jnp.float32),
        pltpu.SemaphoreType.DMA,
    ],
    compiler_params=pltpu.CompilerParams(use_tc_tiling_on_sc=False),
)(_pool_body)


_BN = 512
_NV = (VOCAB + _BN - 1) // _BN


def _mm_body(x_ref, w_ref, b_ref, o_ref):
    o_ref[...] = lax.dot_general(
        x_ref[...], w_ref[...],
        (((1,), (1,)), ((), ())),
        preferred_element_type=jnp.float32,
    ) + b_ref[...]


def _project(pooled, W, b2):
    return pl.pallas_call(
        _mm_body,
        grid=(_NV,),
        in_specs=[
            pl.BlockSpec((BATCH, EMBED), lambda i: (0, 0)),
            pl.BlockSpec((_BN, EMBED), lambda i: (i, 0)),
            pl.BlockSpec((1, _BN), lambda i: (0, i)),
        ],
        out_specs=pl.BlockSpec((BATCH, _BN), lambda i: (0, i)),
        out_shape=jax.ShapeDtypeStruct((BATCH, VOCAB), jnp.float32),
    )(pooled, W, b2)


def kernel(inputs, emb_table, W, b):
    idx = inputs.reshape(-1).astype(jnp.int32)
    pooled = _pool(idx, emb_table)
    return _project(pooled, W, b.reshape(1, VOCAB))
